# Initial kernel scaffold; baseline (speedup 1.0000x reference)
import jax
import jax.numpy as jnp
from jax.experimental import pallas as pl


def kernel(x, edge_index, edge_weight, W1, b1, W2, b2):
    raise NotImplementedError("write your pallas kernel here")


# speedup vs baseline: 9.4460x; 9.4460x over previous
import functools

import jax
import jax.numpy as jnp
from jax import lax
from jax.experimental import pallas as pl
from jax.experimental.pallas import tpu as pltpu
from jax.experimental.pallas import tpu_sc as plsc

N = 10000
D = 256
E_PAD = 163840
L = 16
NW = 32
EPT = E_PAD // NW
CHUNK = 2560
NCHUNK = E_PAD // CHUNK
CPP = 4
NPASS = D // (NW * CPP)


def _wid():
    return lax.axis_index("s") * 2 + lax.axis_index("c")


def _zero_fill(ref, n, unroll):
    zero = jnp.zeros((L,), jnp.float32)

    @plsc.parallel_loop(0, n // L, unroll=unroll)
    def _(i):
        ref[pl.ds(i * L, L)] = zero


@functools.cache
def _build():
    mesh = plsc.VectorSubcoreMesh(core_axis_name="c", subcore_axis_name="s")
    sc_params = pltpu.CompilerParams(needs_layout_passes=False)

    @functools.partial(
        pl.kernel,
        out_type=jax.ShapeDtypeStruct((NW * N,), jnp.float32),
        mesh=mesh,
        compiler_params=sc_params,
        scratch_types=[
            pltpu.VMEM((N,), jnp.float32),
            pltpu.VMEM((EPT,), jnp.int32),
            pltpu.VMEM((EPT,), jnp.float32),
        ],
    )
    def _deg_kernel(dst_hbm, ew_hbm, out_hbm, deg_v, dst_v, ew_v):
        wid = _wid()
        base = wid * EPT
        pltpu.sync_copy(dst_hbm.at[pl.ds(base, EPT)], dst_v)
        pltpu.sync_copy(ew_hbm.at[pl.ds(base, EPT)], ew_v)
        _zero_fill(deg_v, N, 5)

        @plsc.parallel_loop(0, EPT // L, unroll=4)
        def _(k):
            d16 = dst_v[pl.ds(k * L, L)]
            w16 = ew_v[pl.ds(k * L, L)]
            plsc.addupdate_scatter(deg_v, [d16], w16)

        pltpu.sync_copy(deg_v, out_hbm.at[pl.ds(wid * N, N)])

    @functools.partial(
        pl.kernel,
        out_type=jax.ShapeDtypeStruct((3 * E_PAD,), jnp.int32),
        mesh=mesh,
        compiler_params=sc_params,
        scratch_types=[
            pltpu.VMEM((N,), jnp.float32),
            pltpu.VMEM((EPT,), jnp.int32),
            pltpu.VMEM((EPT,), jnp.int32),
            pltpu.VMEM((EPT,), jnp.float32),
            pltpu.VMEM((EPT,), jnp.int32),
        ],
    )
    def _norm_kernel(src_hbm, dst_hbm, ew_hbm, dis_hbm, out_hbm,
                     dis_v, src_v, dst_v, ew_v, nrm_v):
        wid = _wid()
        base = wid * EPT
        pltpu.sync_copy(dis_hbm, dis_v)
        pltpu.sync_copy(src_hbm.at[pl.ds(base, EPT)], src_v)
        pltpu.sync_copy(dst_hbm.at[pl.ds(base, EPT)], dst_v)
        pltpu.sync_copy(ew_hbm.at[pl.ds(base, EPT)], ew_v)

        @plsc.parallel_loop(0, EPT // L, unroll=4)
        def _(k):
            s16 = src_v[pl.ds(k * L, L)]
            d16 = dst_v[pl.ds(k * L, L)]
            w16 = ew_v[pl.ds(k * L, L)]
            a = plsc.load_gather(dis_v, [s16])
            b = plsc.load_gather(dis_v, [d16])
            nrm_v[pl.ds(k * L, L)] = plsc.bitcast(a * w16 * b, jnp.int32)

        for cc in range(2):
            ob = (2 * wid + cc) * 3 * CHUNK
            pltpu.sync_copy(src_v.at[pl.ds(cc * CHUNK, CHUNK)],
                            out_hbm.at[pl.ds(ob, CHUNK)])
            pltpu.sync_copy(dst_v.at[pl.ds(cc * CHUNK, CHUNK)],
                            out_hbm.at[pl.ds(ob + CHUNK, CHUNK)])
            pltpu.sync_copy(nrm_v.at[pl.ds(cc * CHUNK, CHUNK)],
                            out_hbm.at[pl.ds(ob + 2 * CHUNK, CHUNK)])

    @functools.partial(
        pl.kernel,
        out_type=jax.ShapeDtypeStruct((D * N,), jnp.float32),
        mesh=mesh,
        compiler_params=sc_params,
        scratch_types=(
            [pltpu.VMEM((N,), jnp.float32) for _ in range(2 * CPP)]
            + [pltpu.VMEM((3 * CHUNK,), jnp.int32) for _ in range(2)]
            + [pltpu.SemaphoreType.DMA, pltpu.SemaphoreType.DMA]
        ),
    )
    def _feat_scatter_kernel(hT_hbm, edata_hbm, out_hbm,
                             g0, g1, g2, g3, t0, t1, t2, t3,
                             eb0, eb1, sem0, sem1):
        gs = (g0, g1, g2, g3)
        ts = (t0, t1, t2, t3)
        ebs = (eb0, eb1)
        sems = (sem0, sem1)
        wid = _wid()

        def _start(ci, b):
            pltpu.async_copy(
                edata_hbm.at[pl.ds(ci * 3 * CHUNK, 3 * CHUNK)], ebs[b], sems[b])

        def _wait(ci, b):
            pltpu.make_async_copy(
                edata_hbm.at[pl.ds(ci * 3 * CHUNK, 3 * CHUNK)], ebs[b], sems[b]
            ).wait()

        for p in range(NPASS):
            c0 = wid * (CPP * NPASS) + p * CPP
            for j in range(CPP):
                pltpu.sync_copy(hT_hbm.at[pl.ds((c0 + j) * N, N)], gs[j])
                _zero_fill(ts[j], N, 5)
            _start(0, 0)

            def cbody(g, c):
                for b in range(2):
                    ci = g * 2 + b

                    @pl.when(ci + 1 < NCHUNK)
                    def _():
                        _start(ci + 1, 1 - b)

                    _wait(ci, b)
                    ebuf = ebs[b]

                    @plsc.parallel_loop(0, CHUNK // L, unroll=16)
                    def _(k):
                        s16 = ebuf[pl.ds(k * L, L)]
                        d16 = ebuf[pl.ds(CHUNK + k * L, L)]
                        w16 = plsc.bitcast(
                            ebuf[pl.ds(2 * CHUNK + k * L, L)], jnp.float32)
                        for j in range(CPP):
                            v = plsc.load_gather(gs[j], [s16]) * w16
                            plsc.addupdate_scatter(ts[j], [d16], v)
                return c

            lax.fori_loop(0, NCHUNK // 2, cbody, 0)
            for j in range(CPP):
                pltpu.sync_copy(ts[j], out_hbm.at[pl.ds((c0 + j) * N, N)])

    def _prep_body(parts_ref, dis_ref, sc_ref):
        deg = jnp.sum(parts_ref[...], axis=0, keepdims=True) + 1.0
        dis_ref[...] = lax.rsqrt(deg)
        sc_ref[...] = 1.0 / deg

    _prep = pl.pallas_call(
        _prep_body,
        out_shape=[jax.ShapeDtypeStruct((1, N), jnp.float32),
                   jax.ShapeDtypeStruct((1, N), jnp.float32)],
    )

    def _mm_body(w_ref, x_ref, o_ref):
        o_ref[...] = lax.dot_general(
            w_ref[...], x_ref[...], (((0,), (1,)), ((), ())),
            preferred_element_type=jnp.float32)

    _mm1 = pl.pallas_call(
        _mm_body,
        out_shape=jax.ShapeDtypeStruct((D, N), jnp.float32),
    )

    def _epi1_mm2_body(t_ref, h_ref, sc_ref, b_ref, w_ref, o_ref):
        a = t_ref[...] + sc_ref[...] * h_ref[...] + b_ref[...]
        a = jnp.maximum(a, 0.0)
        o_ref[...] = lax.dot_general(
            w_ref[...], a, (((0,), (0,)), ((), ())),
            preferred_element_type=jnp.float32)

    _epi1_mm2 = pl.pallas_call(
        _epi1_mm2_body,
        out_shape=jax.ShapeDtypeStruct((D, N), jnp.float32),
    )

    def _epi2_body(t_ref, h_ref, sc_ref, b_ref, o_ref):
        r = t_ref[...] + sc_ref[...] * h_ref[...] + b_ref[...]
        o_ref[...] = r.T

    _epi2 = pl.pallas_call(
        _epi2_body,
        out_shape=jax.ShapeDtypeStruct((N, D), jnp.float32),
    )

    return (_deg_kernel, _norm_kernel, _feat_scatter_kernel,
            _prep, _mm1, _epi1_mm2, _epi2)


def kernel(x, edge_index, edge_weight, W1, b1, W2, b2):
    (_deg_kernel, _norm_kernel, _feat_scatter_kernel,
     _prep, _mm1, _epi1_mm2, _epi2) = _build()
    src = edge_index[0].astype(jnp.int32)
    dst = edge_index[1].astype(jnp.int32)
    ew = edge_weight.astype(jnp.float32)
    pe = E_PAD - src.shape[0]
    src_p = jnp.pad(src, (0, pe))
    dst_p = jnp.pad(dst, (0, pe))
    ew_p = jnp.pad(ew, (0, pe))

    parts = _deg_kernel(dst_p, ew_p)
    dis2d, sc2d = _prep(parts.reshape(NW, N))
    edata = _norm_kernel(src_p, dst_p, ew_p, dis2d.reshape(-1))

    h1 = _mm1(W1, x)
    t1 = _feat_scatter_kernel(h1.reshape(-1), edata)
    h2 = _epi1_mm2(t1.reshape(D, N), h1, sc2d, b1.reshape(D, 1), W2)
    t2 = _feat_scatter_kernel(h2.reshape(-1), edata)
    return _epi2(t2.reshape(D, N), h2, sc2d, b2.reshape(D, 1))


# speedup vs baseline: 10.5190x; 1.1136x over previous
import functools

import jax
import jax.numpy as jnp
from jax import lax
from jax.experimental import pallas as pl
from jax.experimental.pallas import tpu as pltpu
from jax.experimental.pallas import tpu_sc as plsc

N = 10000
D = 256
E_PAD = 163840
L = 16
NW = 32
EPT = E_PAD // NW
CHUNK = 2560
NCHUNK = E_PAD // CHUNK
CPP = 4
NPASS = D // (NW * CPP)


def _wid():
    return lax.axis_index("s") * 2 + lax.axis_index("c")


def _zero_fill(ref, n, unroll):
    zero = jnp.zeros((L,), jnp.float32)

    @plsc.parallel_loop(0, n // L, unroll=unroll)
    def _(i):
        ref[pl.ds(i * L, L)] = zero


@functools.cache
def _build():
    mesh = plsc.VectorSubcoreMesh(core_axis_name="c", subcore_axis_name="s")
    sc_params = pltpu.CompilerParams(needs_layout_passes=False)

    @functools.partial(
        pl.kernel,
        out_type=jax.ShapeDtypeStruct((NW * N,), jnp.float32),
        mesh=mesh,
        compiler_params=sc_params,
        scratch_types=[
            pltpu.VMEM((N,), jnp.float32),
            pltpu.VMEM((EPT,), jnp.int32),
            pltpu.VMEM((EPT,), jnp.float32),
        ],
    )
    def _deg_kernel(dst_hbm, ew_hbm, out_hbm, deg_v, dst_v, ew_v):
        wid = _wid()
        base = wid * EPT
        pltpu.sync_copy(dst_hbm.at[pl.ds(base, EPT)], dst_v)
        pltpu.sync_copy(ew_hbm.at[pl.ds(base, EPT)], ew_v)
        _zero_fill(deg_v, N, 5)

        @plsc.parallel_loop(0, EPT // L, unroll=4)
        def _(k):
            d16 = dst_v[pl.ds(k * L, L)]
            w16 = ew_v[pl.ds(k * L, L)]
            plsc.addupdate_scatter(deg_v, [d16], w16)

        pltpu.sync_copy(deg_v, out_hbm.at[pl.ds(wid * N, N)])

    @functools.partial(
        pl.kernel,
        out_type=jax.ShapeDtypeStruct((3 * E_PAD,), jnp.int32),
        mesh=mesh,
        compiler_params=sc_params,
        scratch_types=[
            pltpu.VMEM((N,), jnp.float32),
            pltpu.VMEM((EPT,), jnp.int32),
            pltpu.VMEM((EPT,), jnp.int32),
            pltpu.VMEM((EPT,), jnp.float32),
            pltpu.VMEM((EPT,), jnp.int32),
        ],
    )
    def _norm_kernel(src_hbm, dst_hbm, ew_hbm, dis_hbm, out_hbm,
                     dis_v, src_v, dst_v, ew_v, nrm_v):
        wid = _wid()
        base = wid * EPT
        pltpu.sync_copy(dis_hbm, dis_v)
        pltpu.sync_copy(src_hbm.at[pl.ds(base, EPT)], src_v)
        pltpu.sync_copy(dst_hbm.at[pl.ds(base, EPT)], dst_v)
        pltpu.sync_copy(ew_hbm.at[pl.ds(base, EPT)], ew_v)

        @plsc.parallel_loop(0, EPT // L, unroll=4)
        def _(k):
            s16 = src_v[pl.ds(k * L, L)]
            d16 = dst_v[pl.ds(k * L, L)]
            w16 = ew_v[pl.ds(k * L, L)]
            a = plsc.load_gather(dis_v, [s16])
            b = plsc.load_gather(dis_v, [d16])
            nrm_v[pl.ds(k * L, L)] = plsc.bitcast(a * w16 * b, jnp.int32)

        for cc in range(2):
            ob = (2 * wid + cc) * 3 * CHUNK
            pltpu.sync_copy(src_v.at[pl.ds(cc * CHUNK, CHUNK)],
                            out_hbm.at[pl.ds(ob, CHUNK)])
            pltpu.sync_copy(dst_v.at[pl.ds(cc * CHUNK, CHUNK)],
                            out_hbm.at[pl.ds(ob + CHUNK, CHUNK)])
            pltpu.sync_copy(nrm_v.at[pl.ds(cc * CHUNK, CHUNK)],
                            out_hbm.at[pl.ds(ob + 2 * CHUNK, CHUNK)])

    @functools.partial(
        pl.kernel,
        out_type=jax.ShapeDtypeStruct((D * N,), jnp.float32),
        mesh=mesh,
        compiler_params=sc_params,
        scratch_types=(
            [pltpu.VMEM((N,), jnp.int32) for _ in range(2)]
            + [pltpu.VMEM((N,), jnp.float32) for _ in range(4)]
            + [pltpu.VMEM((3 * CHUNK,), jnp.int32) for _ in range(2)]
            + [pltpu.SemaphoreType.DMA, pltpu.SemaphoreType.DMA]
        ),
    )
    def _feat_scatter_kernel(hP_hbm, edata_hbm, out_hbm,
                             gp0, gp1, t00, t01, t10, t11,
                             eb0, eb1, sem0, sem1):
        gps = (gp0, gp1)
        tss = ((t00, t01), (t10, t11))
        ebs = (eb0, eb1)
        sems = (sem0, sem1)
        wid = _wid()

        def _start(ci, b):
            pltpu.async_copy(
                edata_hbm.at[pl.ds(ci * 3 * CHUNK, 3 * CHUNK)], ebs[b], sems[b])

        def _wait(ci, b):
            pltpu.make_async_copy(
                edata_hbm.at[pl.ds(ci * 3 * CHUNK, 3 * CHUNK)], ebs[b], sems[b]
            ).wait()

        for p in range(NPASS):
            pr0 = wid * (2 * NPASS) + p * 2
            for j in range(2):
                pltpu.sync_copy(hP_hbm.at[pl.ds((pr0 + j) * N, N)], gps[j])
                _zero_fill(tss[j][0], N, 5)
                _zero_fill(tss[j][1], N, 5)
            _start(0, 0)

            def cbody(g, c):
                for b in range(2):
                    ci = g * 2 + b

                    @pl.when(ci + 1 < NCHUNK)
                    def _():
                        _start(ci + 1, 1 - b)

                    _wait(ci, b)
                    ebuf = ebs[b]

                    @plsc.parallel_loop(0, CHUNK // L, unroll=16)
                    def _(k):
                        s16 = ebuf[pl.ds(k * L, L)]
                        d16 = ebuf[pl.ds(CHUNK + k * L, L)]
                        w16 = plsc.bitcast(
                            ebuf[pl.ds(2 * CHUNK + k * L, L)], jnp.float32)
                        for j in range(2):
                            pk = plsc.load_gather(gps[j], [s16])
                            lo, hi = plsc.unpack(
                                plsc.bitcast(pk, jnp.bfloat16),
                                format=plsc.PackFormat.INTERLEAVED)
                            plsc.addupdate_scatter(tss[j][0], [d16], lo * w16)
                            plsc.addupdate_scatter(tss[j][1], [d16], hi * w16)
                return c

            lax.fori_loop(0, NCHUNK // 2, cbody, 0)
            for j in range(2):
                pltpu.sync_copy(tss[j][0], out_hbm.at[pl.ds((pr0 + j) * N, N)])
                pltpu.sync_copy(tss[j][1],
                                out_hbm.at[pl.ds((pr0 + j + D // 2) * N, N)])

    def _prep_body(parts_ref, dis_ref, sc_ref):
        deg = jnp.sum(parts_ref[...], axis=0, keepdims=True) + 1.0
        dis_ref[...] = lax.rsqrt(deg)
        sc_ref[...] = 1.0 / deg

    _prep = pl.pallas_call(
        _prep_body,
        out_shape=[jax.ShapeDtypeStruct((1, N), jnp.float32),
                   jax.ShapeDtypeStruct((1, N), jnp.float32)],
    )

    def _pack_pairs(h):
        lo = lax.bitcast_convert_type(
            h[:D // 2].astype(jnp.bfloat16), jnp.uint16).astype(jnp.uint32)
        hi = lax.bitcast_convert_type(
            h[D // 2:].astype(jnp.bfloat16), jnp.uint16).astype(jnp.uint32)
        return lax.bitcast_convert_type(lo | (hi << 16), jnp.int32)

    def _mm_body(w_ref, x_ref, o_ref, p_ref):
        h = lax.dot_general(
            w_ref[...], x_ref[...], (((0,), (1,)), ((), ())),
            preferred_element_type=jnp.float32)
        o_ref[...] = h
        p_ref[...] = _pack_pairs(h)

    _mm1 = pl.pallas_call(
        _mm_body,
        out_shape=[jax.ShapeDtypeStruct((D, N), jnp.float32),
                   jax.ShapeDtypeStruct((D // 2, N), jnp.int32)],
    )

    def _epi1_mm2_body(t_ref, h_ref, sc_ref, b_ref, w_ref, o_ref, p_ref):
        a = t_ref[...] + sc_ref[...] * h_ref[...] + b_ref[...]
        a = jnp.maximum(a, 0.0)
        h2 = lax.dot_general(
            w_ref[...], a, (((0,), (0,)), ((), ())),
            preferred_element_type=jnp.float32)
        o_ref[...] = h2
        p_ref[...] = _pack_pairs(h2)

    _epi1_mm2 = pl.pallas_call(
        _epi1_mm2_body,
        out_shape=[jax.ShapeDtypeStruct((D, N), jnp.float32),
                   jax.ShapeDtypeStruct((D // 2, N), jnp.int32)],
    )

    def _epi2_body(t_ref, h_ref, sc_ref, b_ref, o_ref):
        r = t_ref[...] + sc_ref[...] * h_ref[...] + b_ref[...]
        o_ref[...] = r.T

    _epi2 = pl.pallas_call(
        _epi2_body,
        out_shape=jax.ShapeDtypeStruct((N, D), jnp.float32),
    )

    return (_deg_kernel, _norm_kernel, _feat_scatter_kernel,
            _prep, _mm1, _epi1_mm2, _epi2)


def kernel(x, edge_index, edge_weight, W1, b1, W2, b2):
    (_deg_kernel, _norm_kernel, _feat_scatter_kernel,
     _prep, _mm1, _epi1_mm2, _epi2) = _build()
    src = edge_index[0].astype(jnp.int32)
    dst = edge_index[1].astype(jnp.int32)
    ew = edge_weight.astype(jnp.float32)
    pe = E_PAD - src.shape[0]
    src_p = jnp.pad(src, (0, pe))
    dst_p = jnp.pad(dst, (0, pe))
    ew_p = jnp.pad(ew, (0, pe))

    parts = _deg_kernel(dst_p, ew_p)
    dis2d, sc2d = _prep(parts.reshape(NW, N))
    edata = _norm_kernel(src_p, dst_p, ew_p, dis2d.reshape(-1))

    h1, h1p = _mm1(W1, x)
    t1 = _feat_scatter_kernel(h1p.reshape(-1), edata)
    h2, h2p = _epi1_mm2(t1.reshape(D, N), h1, sc2d, b1.reshape(D, 1), W2)
    t2 = _feat_scatter_kernel(h2p.reshape(-1), edata)
    return _epi2(t2.reshape(D, N), h2, sc2d, b2.reshape(D, 1))
